# baseline (speedup 1.0000x reference)
import jax
import jax.numpy as jnp
from jax.experimental import pallas as pl
from jax.experimental.pallas import tpu as pltpu

N = 10000
K = 16
D_IN = 8
D_MODEL = 128
NUM_CLASSES = 16
NPAD = 10240
CHUNK = 5008
NBLK = 2 * K
NCH = 2
NSTEP = NBLK * NCH
E = 2 * K * N - K * (K + 1)


def _node_kernel(x_ref, w1_ref, b1_ref, w2_ref, b2_ref, wt_ref, wb_ref,
                 a_ref, b_ref):
    x = x_ref[...]
    ii = jax.lax.broadcasted_iota(jnp.int32, (NPAD, 1), 0).astype(jnp.float32)
    deg = (jnp.minimum(ii, float(K)) +
           jnp.minimum(float(N - 1) - ii, float(K)) + 1.0)
    dinv = jnp.where(ii < float(N), jax.lax.rsqrt(jnp.maximum(deg, 1.0)), 0.0)

    def shift_up(v, s):
        return jnp.concatenate([v[s:], jnp.zeros((s, D_MODEL), jnp.float32)], 0)

    def shift_down(v, s):
        return jnp.concatenate([jnp.zeros((s, D_MODEL), jnp.float32), v[:-s]], 0)

    def conv(h, bias):
        z = dinv * h
        zp = shift_down(z, K)
        p = zp
        for s in (1, 2, 4, 8, 16):
            p = p + shift_up(p, s)
        w = p + shift_up(zp, 2 * K)
        return jax.nn.relu(dinv * w + bias)

    h1 = jnp.dot(x, w1_ref[...], preferred_element_type=jnp.float32)
    x1 = conv(h1, b1_ref[...])
    h2 = jnp.dot(x1, w2_ref[...], preferred_element_type=jnp.float32)
    x2 = conv(h2, b2_ref[...])
    a_ref[...] = jnp.dot(x2, wt_ref[...], preferred_element_type=jnp.float32)
    b_ref[...] = jnp.dot(x2, wb_ref[...], preferred_element_type=jnp.float32)


def _edge_kernel(a_ref, b_ref, bs_ref, idx_ref, bl1_ref, wf_ref, bf_ref,
                 probs_ref, bbox_ref, ip_ref,
                 sp_ref, sba_ref, sbb_ref, sia_ref, sib_ref,
                 p_sem, st_sem, ob_sem, oi_sem):
    blk = pl.program_id(0)
    c = pl.program_id(1)
    step = blk * NCH + c
    slot = jax.lax.rem(step, 2)

    q = blk // 2
    off = q + 1
    rev = jax.lax.rem(blk, 2)
    lblk = N - off
    start_blk = 2 * q * N - q * (q + 1) + rev * lblk
    cs = jnp.minimum(c * CHUNK, lblk - CHUNK)
    row0 = start_blk + cs
    p_src = cs + rev * off
    p_dst = cs + (1 - rev) * off

    def probs_copy(s):
        return pltpu.make_async_copy(
            sp_ref.at[s], probs_ref.at[pl.ds(row0, CHUNK)], p_sem.at[s])

    def stage_copies(s):
        return (
            pltpu.make_async_copy(bs_ref.at[pl.ds(p_src, CHUNK), :],
                                  sba_ref.at[s], st_sem.at[s]),
            pltpu.make_async_copy(bs_ref.at[pl.ds(p_dst, CHUNK), :],
                                  sbb_ref.at[s], st_sem.at[s]),
            pltpu.make_async_copy(idx_ref.at[pl.ds(p_src, CHUNK), :],
                                  sia_ref.at[s], st_sem.at[s]),
            pltpu.make_async_copy(idx_ref.at[pl.ds(p_dst, CHUNK), :],
                                  sib_ref.at[s], st_sem.at[s]),
        )

    def out_copies(s):
        return (
            pltpu.make_async_copy(
                sba_ref.at[s],
                bbox_ref.at[pl.ds(row0, CHUNK), 0], ob_sem.at[s]),
            pltpu.make_async_copy(
                sbb_ref.at[s],
                bbox_ref.at[pl.ds(row0, CHUNK), 1], ob_sem.at[s]),
            pltpu.make_async_copy(
                sia_ref.at[s],
                ip_ref.at[pl.ds(row0, CHUNK), 0], oi_sem.at[s]),
            pltpu.make_async_copy(
                sib_ref.at[s],
                ip_ref.at[pl.ds(row0, CHUNK), 1], oi_sem.at[s]),
        )

    @pl.when(step >= 2)
    def _():
        probs_copy(slot).wait()
        for cp in out_copies(slot):
            cp.wait()

    for cp in stage_copies(slot):
        cp.start()

    a = a_ref[pl.ds(p_src, CHUNK), :]
    b = b_ref[pl.ds(p_dst, CHUNK), :]
    h = jax.nn.relu(a + b + bl1_ref[...])
    logits = jnp.dot(h, wf_ref[...], preferred_element_type=jnp.float32)
    logits = logits + bf_ref[...]
    m = jnp.max(logits, axis=-1, keepdims=True)
    lse = jnp.log(jnp.sum(jnp.exp(logits - m), axis=-1, keepdims=True)) + m
    sp_ref[slot] = logits - lse

    for cp in stage_copies(slot):
        cp.wait()
    for cp in out_copies(slot):
        cp.start()
    probs_copy(slot).start()

    @pl.when(step == NSTEP - 1)
    def _():
        for s in (slot, 1 - slot):
            probs_copy(s).wait()
            for cp in out_copies(s):
                cp.wait()


def kernel(feature_vec, bboxes, bbox_indices, W1, b1, W2, b2, Wl1, bl1, Wf, bf):
    centers = (bboxes[:, 0:2] + bboxes[:, 2:4]) * 0.5
    keyv = centers[:, 0] + 1e-3 * centers[:, 1]
    order = jnp.argsort(keyv)

    x_s = feature_vec[order]
    bs = bboxes[order]
    idx_s = bbox_indices[order].astype(jnp.int32)

    pad = NPAD - N
    x_s = jnp.pad(x_s, ((0, pad), (0, 0)))
    bs = jnp.pad(bs, ((0, pad), (0, 0)))
    idx_s = jnp.pad(idx_s, (0, pad)).reshape(NPAD, 1)

    full = lambda shape: pl.BlockSpec(shape, lambda: tuple(0 for _ in shape))

    A, B = pl.pallas_call(
        _node_kernel,
        out_shape=(
            jax.ShapeDtypeStruct((NPAD, D_MODEL), jnp.float32),
            jax.ShapeDtypeStruct((NPAD, D_MODEL), jnp.float32),
        ),
        in_specs=[full((NPAD, D_IN)), full((D_IN, D_MODEL)),
                  full((1, D_MODEL)), full((D_MODEL, D_MODEL)),
                  full((1, D_MODEL)), full((D_MODEL, D_MODEL)),
                  full((D_MODEL, D_MODEL))],
        out_specs=(full((NPAD, D_MODEL)), full((NPAD, D_MODEL))),
    )(x_s, W1, b1.reshape(1, -1), W2, b2.reshape(1, -1),
      Wl1[:D_MODEL], Wl1[D_MODEL:])

    cfull = lambda shape: pl.BlockSpec(shape, lambda b, c: tuple(0 for _ in shape))
    anyspec = pl.BlockSpec(memory_space=pl.MemorySpace.ANY)
    probs, bbox_pairs, bbox_index_pairs = pl.pallas_call(
        _edge_kernel,
        grid=(NBLK, NCH),
        out_shape=(
            jax.ShapeDtypeStruct((E, NUM_CLASSES), jnp.float32),
            jax.ShapeDtypeStruct((E, 2, 4), jnp.float32),
            jax.ShapeDtypeStruct((E, 2, 1), jnp.int32),
        ),
        in_specs=[cfull((NPAD, D_MODEL)), cfull((NPAD, D_MODEL)),
                  anyspec, anyspec,
                  cfull((1, D_MODEL)), cfull((D_MODEL, NUM_CLASSES)),
                  cfull((1, NUM_CLASSES))],
        out_specs=(anyspec, anyspec, anyspec),
        scratch_shapes=[
            pltpu.VMEM((2, CHUNK, NUM_CLASSES), jnp.float32),
            pltpu.VMEM((2, CHUNK, 4), jnp.float32),
            pltpu.VMEM((2, CHUNK, 4), jnp.float32),
            pltpu.VMEM((2, CHUNK, 1), jnp.int32),
            pltpu.VMEM((2, CHUNK, 1), jnp.int32),

            pltpu.SemaphoreType.DMA((2,)),
            pltpu.SemaphoreType.DMA((2,)),
            pltpu.SemaphoreType.DMA((2,)),
            pltpu.SemaphoreType.DMA((2,)),
        ],
    )(A, B, bs, idx_s, bl1.reshape(1, -1), Wf, bf.reshape(1, -1))

    bbox_pairs = bbox_pairs.reshape(E, 8)
    bbox_index_pairs = bbox_index_pairs.reshape(E, 2)
    return (probs, bbox_pairs, bbox_index_pairs)


# speedup vs baseline: 2.6180x; 2.6180x over previous
import jax
import jax.numpy as jnp
from jax.experimental import pallas as pl
from jax.experimental.pallas import tpu as pltpu

N = 10000
K = 16
D_IN = 8
D_MODEL = 128
NUM_CLASSES = 16
NPAD = 10240
CHUNK = 5008
NBLK = 2 * K
NCH = 2
NSTEP = NBLK * NCH
E = 2 * K * N - K * (K + 1)


def _node_kernel(x_ref, w1_ref, b1_ref, w2_ref, b2_ref, wt_ref, wb_ref,
                 a_ref, b_ref):
    x = x_ref[...]
    ii = jax.lax.broadcasted_iota(jnp.int32, (NPAD, 1), 0).astype(jnp.float32)
    deg = (jnp.minimum(ii, float(K)) +
           jnp.minimum(float(N - 1) - ii, float(K)) + 1.0)
    dinv = jnp.where(ii < float(N), jax.lax.rsqrt(jnp.maximum(deg, 1.0)), 0.0)

    def shift_up(v, s):
        return jnp.concatenate([v[s:], jnp.zeros((s, D_MODEL), jnp.float32)], 0)

    def shift_down(v, s):
        return jnp.concatenate([jnp.zeros((s, D_MODEL), jnp.float32), v[:-s]], 0)

    def conv(h, bias):
        z = dinv * h
        zp = shift_down(z, K)
        p = zp
        for s in (1, 2, 4, 8, 16):
            p = p + shift_up(p, s)
        w = p + shift_up(zp, 2 * K)
        return jax.nn.relu(dinv * w + bias)

    h1 = jnp.dot(x, w1_ref[...], preferred_element_type=jnp.float32)
    x1 = conv(h1, b1_ref[...])
    h2 = jnp.dot(x1, w2_ref[...], preferred_element_type=jnp.float32)
    x2 = conv(h2, b2_ref[...])
    a_ref[...] = jnp.dot(x2, wt_ref[...], preferred_element_type=jnp.float32)
    b_ref[...] = jnp.dot(x2, wb_ref[...], preferred_element_type=jnp.float32)


def _edge_kernel(a_ref, b_ref, bs_ref, idx_ref, bl1_ref, wf_ref, bf_ref,
                 probs_ref, bbox_ref, ip_ref,
                 sp_ref, sb8_ref, si2_ref,
                 p_sem, ob_sem, oi_sem):
    blk = pl.program_id(0)
    c = pl.program_id(1)
    step = blk * NCH + c
    slot = jax.lax.rem(step, 2)

    q = blk // 2
    off = q + 1
    rev = jax.lax.rem(blk, 2)
    lblk = N - off
    start_blk = 2 * q * N - q * (q + 1) + rev * lblk
    cs = jnp.minimum(c * CHUNK, lblk - CHUNK)
    row0 = start_blk + cs
    p_src = cs + rev * off
    p_dst = cs + (1 - rev) * off

    def probs_copy(s):
        return pltpu.make_async_copy(
            sp_ref.at[s], probs_ref.at[pl.ds(row0, CHUNK)], p_sem.at[s])

    def out_copies(s):
        return (
            pltpu.make_async_copy(
                sb8_ref.at[s], bbox_ref.at[pl.ds(row0, CHUNK)], ob_sem.at[s]),
            pltpu.make_async_copy(
                si2_ref.at[s], ip_ref.at[pl.ds(row0, CHUNK)], oi_sem.at[s]),
        )

    @pl.when(step >= 2)
    def _():
        probs_copy(slot).wait()
        for cp in out_copies(slot):
            cp.wait()

    a = a_ref[pl.ds(p_src, CHUNK), :]
    b = b_ref[pl.ds(p_dst, CHUNK), :]
    h = jax.nn.relu(a + b + bl1_ref[...])
    logits = jnp.dot(h, wf_ref[...], preferred_element_type=jnp.float32)
    logits = logits + bf_ref[...]
    m = jnp.max(logits, axis=-1, keepdims=True)
    lse = jnp.log(jnp.sum(jnp.exp(logits - m), axis=-1, keepdims=True)) + m
    sp_ref[slot] = logits - lse
    sb8_ref[slot] = jnp.concatenate(
        [bs_ref[pl.ds(p_src, CHUNK), :], bs_ref[pl.ds(p_dst, CHUNK), :]],
        axis=1)
    si2_ref[slot] = jnp.concatenate(
        [idx_ref[pl.ds(p_src, CHUNK), :], idx_ref[pl.ds(p_dst, CHUNK), :]],
        axis=1)

    for cp in out_copies(slot):
        cp.start()
    probs_copy(slot).start()

    @pl.when(step == NSTEP - 1)
    def _():
        for s in (slot, 1 - slot):
            probs_copy(s).wait()
            for cp in out_copies(s):
                cp.wait()


def kernel(feature_vec, bboxes, bbox_indices, W1, b1, W2, b2, Wl1, bl1, Wf, bf):
    centers = (bboxes[:, 0:2] + bboxes[:, 2:4]) * 0.5
    keyv = centers[:, 0] + 1e-3 * centers[:, 1]
    order = jnp.argsort(keyv)

    x_s = feature_vec[order]
    bs = bboxes[order]
    idx_s = bbox_indices[order].astype(jnp.int32)

    pad = NPAD - N
    x_s = jnp.pad(x_s, ((0, pad), (0, 0)))
    bs = jnp.pad(bs, ((0, pad), (0, 0)))
    idx_s = jnp.pad(idx_s, (0, pad)).reshape(NPAD, 1)

    full = lambda shape: pl.BlockSpec(shape, lambda: tuple(0 for _ in shape))

    A, B = pl.pallas_call(
        _node_kernel,
        out_shape=(
            jax.ShapeDtypeStruct((NPAD, D_MODEL), jnp.float32),
            jax.ShapeDtypeStruct((NPAD, D_MODEL), jnp.float32),
        ),
        in_specs=[full((NPAD, D_IN)), full((D_IN, D_MODEL)),
                  full((1, D_MODEL)), full((D_MODEL, D_MODEL)),
                  full((1, D_MODEL)), full((D_MODEL, D_MODEL)),
                  full((D_MODEL, D_MODEL))],
        out_specs=(full((NPAD, D_MODEL)), full((NPAD, D_MODEL))),
    )(x_s, W1, b1.reshape(1, -1), W2, b2.reshape(1, -1),
      Wl1[:D_MODEL], Wl1[D_MODEL:])

    cfull = lambda shape: pl.BlockSpec(shape, lambda b, c: tuple(0 for _ in shape))
    anyspec = pl.BlockSpec(memory_space=pl.MemorySpace.ANY)
    probs, bbox_pairs, bbox_index_pairs = pl.pallas_call(
        _edge_kernel,
        grid=(NBLK, NCH),
        out_shape=(
            jax.ShapeDtypeStruct((E, NUM_CLASSES), jnp.float32),
            jax.ShapeDtypeStruct((E, 8), jnp.float32),
            jax.ShapeDtypeStruct((E, 2), jnp.int32),
        ),
        in_specs=[cfull((NPAD, D_MODEL)), cfull((NPAD, D_MODEL)),
                  cfull((NPAD, 4)), cfull((NPAD, 1)),
                  cfull((1, D_MODEL)), cfull((D_MODEL, NUM_CLASSES)),
                  cfull((1, NUM_CLASSES))],
        out_specs=(anyspec, anyspec, anyspec),
        scratch_shapes=[
            pltpu.VMEM((2, CHUNK, NUM_CLASSES), jnp.float32),
            pltpu.VMEM((2, CHUNK, 8), jnp.float32),
            pltpu.VMEM((2, CHUNK, 2), jnp.int32),
            pltpu.SemaphoreType.DMA((2,)),
            pltpu.SemaphoreType.DMA((2,)),
            pltpu.SemaphoreType.DMA((2,)),
        ],
    )(A, B, bs, idx_s, bl1.reshape(1, -1), Wf, bf.reshape(1, -1))

    return (probs, bbox_pairs, bbox_index_pairs)


# speedup vs baseline: 2.6548x; 1.0140x over previous
import jax
import jax.numpy as jnp
from jax.experimental import pallas as pl
from jax.experimental.pallas import tpu as pltpu

N = 10000
K = 16
D_IN = 8
D_MODEL = 128
NUM_CLASSES = 16
NPAD = 10240
CHUNK = 5008
NBLK = 2 * K
NCH = 2
NSTEP = NBLK * NCH
E = 2 * K * N - K * (K + 1)


def _node_kernel(x_ref, w1_ref, b1_ref, w2_ref, b2_ref, wt_ref, wb_ref,
                 a_ref, b_ref):
    x = x_ref[...]
    ii = jax.lax.broadcasted_iota(jnp.int32, (NPAD, 1), 0).astype(jnp.float32)
    deg = (jnp.minimum(ii, float(K)) +
           jnp.minimum(float(N - 1) - ii, float(K)) + 1.0)
    dinv = jnp.where(ii < float(N), jax.lax.rsqrt(jnp.maximum(deg, 1.0)), 0.0)

    def shift_up(v, s):
        return jnp.concatenate([v[s:], jnp.zeros((s, D_MODEL), jnp.float32)], 0)

    def shift_down(v, s):
        return jnp.concatenate([jnp.zeros((s, D_MODEL), jnp.float32), v[:-s]], 0)

    def conv(h, bias):
        z = dinv * h
        zp = shift_down(z, K)
        p = zp
        for s in (1, 2, 4, 8, 16):
            p = p + shift_up(p, s)
        w = p + shift_up(zp, 2 * K)
        return jax.nn.relu(dinv * w + bias)

    h1 = jnp.dot(x, w1_ref[...], preferred_element_type=jnp.float32)
    x1 = conv(h1, b1_ref[...])
    h2 = jnp.dot(x1, w2_ref[...], preferred_element_type=jnp.float32)
    x2 = conv(h2, b2_ref[...])
    a_ref[...] = jnp.dot(x2, wt_ref[...], preferred_element_type=jnp.float32)
    b_ref[...] = jnp.dot(x2, wb_ref[...], preferred_element_type=jnp.float32)


def _edge_kernel(a_ref, b_ref, bs_ref, idx_ref, bl1_ref, wf_ref, bf_ref,
                 probs_ref, bbox_ref, ip_ref,
                 sp_ref, sb8_ref, si2_ref,
                 p_sem, ob_sem, oi_sem):
    blk = pl.program_id(0)
    c = pl.program_id(1)
    step = blk * NCH + c
    slot = jax.lax.rem(step, 2)

    q = blk // 2
    off = q + 1
    rev = jax.lax.rem(blk, 2)
    lblk = N - off
    start_blk = 2 * q * N - q * (q + 1) + rev * lblk
    cs = jnp.minimum(c * CHUNK, lblk - CHUNK)
    row0 = start_blk + cs
    p_src = cs + rev * off
    p_dst = cs + (1 - rev) * off

    def probs_copy(s):
        return pltpu.make_async_copy(
            sp_ref.at[s], probs_ref.at[pl.ds(row0, CHUNK)], p_sem.at[s])

    def out_copies(s):
        return (
            pltpu.make_async_copy(
                sb8_ref.at[s], bbox_ref.at[pl.ds(row0, CHUNK)], ob_sem.at[s]),
            pltpu.make_async_copy(
                si2_ref.at[s], ip_ref.at[pl.ds(row0, CHUNK)], oi_sem.at[s]),
        )

    @pl.when(step >= 2)
    def _():
        probs_copy(slot).wait()
        for cp in out_copies(slot):
            cp.wait()

    a = a_ref[pl.ds(p_src, CHUNK), :]
    b = b_ref[pl.ds(p_dst, CHUNK), :]
    h = jax.nn.relu(a + b + bl1_ref[...])
    logits = jnp.dot(h.astype(jnp.bfloat16), wf_ref[...].astype(jnp.bfloat16),
                     preferred_element_type=jnp.float32)
    logits = logits + bf_ref[...]
    m = jnp.max(logits, axis=-1, keepdims=True)
    lse = jnp.log(jnp.sum(jnp.exp(logits - m), axis=-1, keepdims=True)) + m
    sp_ref[slot] = logits - lse
    sb8_ref[slot] = jnp.concatenate(
        [bs_ref[pl.ds(p_src, CHUNK), :], bs_ref[pl.ds(p_dst, CHUNK), :]],
        axis=1)
    si2_ref[slot] = jnp.concatenate(
        [idx_ref[pl.ds(p_src, CHUNK), :], idx_ref[pl.ds(p_dst, CHUNK), :]],
        axis=1)

    for cp in out_copies(slot):
        cp.start()
    probs_copy(slot).start()

    @pl.when(step == NSTEP - 1)
    def _():
        for s in (slot, 1 - slot):
            probs_copy(s).wait()
            for cp in out_copies(s):
                cp.wait()


def kernel(feature_vec, bboxes, bbox_indices, W1, b1, W2, b2, Wl1, bl1, Wf, bf):
    centers = (bboxes[:, 0:2] + bboxes[:, 2:4]) * 0.5
    keyv = centers[:, 0] + 1e-3 * centers[:, 1]
    order = jnp.argsort(keyv)

    x_s = feature_vec[order]
    bs = bboxes[order]
    idx_s = bbox_indices[order].astype(jnp.int32)

    pad = NPAD - N
    x_s = jnp.pad(x_s, ((0, pad), (0, 0)))
    bs = jnp.pad(bs, ((0, pad), (0, 0)))
    idx_s = jnp.pad(idx_s, (0, pad)).reshape(NPAD, 1)

    full = lambda shape: pl.BlockSpec(shape, lambda: tuple(0 for _ in shape))

    A, B = pl.pallas_call(
        _node_kernel,
        out_shape=(
            jax.ShapeDtypeStruct((NPAD, D_MODEL), jnp.float32),
            jax.ShapeDtypeStruct((NPAD, D_MODEL), jnp.float32),
        ),
        in_specs=[full((NPAD, D_IN)), full((D_IN, D_MODEL)),
                  full((1, D_MODEL)), full((D_MODEL, D_MODEL)),
                  full((1, D_MODEL)), full((D_MODEL, D_MODEL)),
                  full((D_MODEL, D_MODEL))],
        out_specs=(full((NPAD, D_MODEL)), full((NPAD, D_MODEL))),
    )(x_s, W1, b1.reshape(1, -1), W2, b2.reshape(1, -1),
      Wl1[:D_MODEL], Wl1[D_MODEL:])

    cfull = lambda shape: pl.BlockSpec(shape, lambda b, c: tuple(0 for _ in shape))
    anyspec = pl.BlockSpec(memory_space=pl.MemorySpace.ANY)
    probs, bbox_pairs, bbox_index_pairs = pl.pallas_call(
        _edge_kernel,
        grid=(NBLK, NCH),
        out_shape=(
            jax.ShapeDtypeStruct((E, NUM_CLASSES), jnp.float32),
            jax.ShapeDtypeStruct((E, 8), jnp.float32),
            jax.ShapeDtypeStruct((E, 2), jnp.int32),
        ),
        in_specs=[cfull((NPAD, D_MODEL)), cfull((NPAD, D_MODEL)),
                  cfull((NPAD, 4)), cfull((NPAD, 1)),
                  cfull((1, D_MODEL)), cfull((D_MODEL, NUM_CLASSES)),
                  cfull((1, NUM_CLASSES))],
        out_specs=(anyspec, anyspec, anyspec),
        scratch_shapes=[
            pltpu.VMEM((2, CHUNK, NUM_CLASSES), jnp.float32),
            pltpu.VMEM((2, CHUNK, 8), jnp.float32),
            pltpu.VMEM((2, CHUNK, 2), jnp.int32),
            pltpu.SemaphoreType.DMA((2,)),
            pltpu.SemaphoreType.DMA((2,)),
            pltpu.SemaphoreType.DMA((2,)),
        ],
    )(A, B, bs, idx_s, bl1.reshape(1, -1), Wf, bf.reshape(1, -1))

    return (probs, bbox_pairs, bbox_index_pairs)
